# baseline (speedup 1.0000x reference)
import functools

import jax
import jax.numpy as jnp
from jax import lax
from jax.experimental import pallas as pl
from jax.experimental.pallas import tpu as pltpu
from jax.experimental.pallas import tpu_sc as plsc

N_IN = 16384
N_OUT = 128
B = 1024
KBLK = 2048

SC_COLS = 2048
TC_COLS = N_IN - SC_COLS
NB = TC_COLS // KBLK

NW = 32
ROWS_W = B // NW
CH = 1024
NCH = SC_COLS // CH
GP_CH = CH // 16


def _tc_main_kernel(x_ref, w_ref, idx_ref, out_ref, acc_ref):
    k = pl.program_id(0)
    idxv = idx_ref[0, 0, :]
    onehot = jnp.where(
        idxv[:, None] == jax.lax.broadcasted_iota(jnp.int32, (KBLK, N_OUT), 1),
        1.0,
        0.0,
    ).astype(jnp.bfloat16)
    xw = (x_ref[...] * w_ref[0, :][None, :]).astype(jnp.bfloat16)
    contrib = jnp.dot(xw, onehot, preferred_element_type=jnp.float32)

    @pl.when(k == 0)
    def _init():
        acc_ref[...] = contrib

    @pl.when(k > 0)
    def _acc():
        acc_ref[...] += contrib

    @pl.when(k == NB - 1)
    def _flush():
        out_ref[...] = acc_ref[...]


def _tc_main(x, weight, idx3):
    return pl.pallas_call(
        _tc_main_kernel,
        grid=(NB,),
        in_specs=[
            pl.BlockSpec((B, KBLK), lambda k: (0, k)),
            pl.BlockSpec((1, KBLK), lambda k: (0, k)),
            pl.BlockSpec((1, 1, KBLK), lambda k: (k, 0, 0)),
        ],
        out_specs=pl.BlockSpec((B, N_OUT), lambda k: (0, 0)),
        out_shape=jax.ShapeDtypeStruct((B, N_OUT), jnp.float32),
        scratch_shapes=[pltpu.VMEM((B, N_OUT), jnp.float32)],
    )(x, weight, idx3)


def _sc_partial(x, wsc, idxsc):
    mesh = plsc.VectorSubcoreMesh(core_axis_name="c", subcore_axis_name="s")

    @functools.partial(
        pl.kernel,
        mesh=mesh,
        out_type=jax.ShapeDtypeStruct((B * N_OUT,), jnp.float32),
        compiler_params=pltpu.CompilerParams(needs_layout_passes=False),
        scratch_types=[
            pltpu.VMEM((ROWS_W, CH), jnp.float32),
            pltpu.VMEM((SC_COLS,), jnp.float32),
            pltpu.VMEM((SC_COLS,), jnp.int32),
            pltpu.VMEM((ROWS_W * N_OUT,), jnp.float32),
        ],
    )
    def sc_kernel(x_hbm, w_hbm, idx_hbm, out_hbm, xbuf, wbuf, cbuf, acc):
        wid = lax.axis_index("s") * 2 + lax.axis_index("c")
        row0 = wid * ROWS_W
        pltpu.sync_copy(w_hbm, wbuf)
        pltpu.sync_copy(idx_hbm, cbuf)

        def zero_blk(i, _):
            acc[pl.ds(i * 16, 16)] = jnp.zeros((16,), jnp.float32)
            return 0

        lax.fori_loop(0, ROWS_W * N_OUT // 16, zero_blk, 0)

        def chunk_body(ci, _):
            pltpu.sync_copy(
                x_hbm.at[pl.ds(row0, ROWS_W), pl.ds(TC_COLS + ci * CH, CH)],
                xbuf,
            )

            def group_body(g, _):
                col0 = ci * CH + g * 16
                wv = wbuf[pl.ds(col0, 16)]
                capv = cbuf[pl.ds(col0, 16)]

                def row_body(r, _):
                    xv = xbuf[r, pl.ds(g * 16, 16)]
                    v = xv * wv
                    dst = capv + r * N_OUT
                    plsc.addupdate_scatter(acc, [dst], v)
                    return 0

                lax.fori_loop(0, ROWS_W, row_body, 0)
                return 0

            lax.fori_loop(0, GP_CH, group_body, 0)
            return 0

        lax.fori_loop(0, NCH, chunk_body, 0)
        pltpu.sync_copy(acc, out_hbm.at[pl.ds(row0 * N_OUT, ROWS_W * N_OUT)])

    return sc_kernel(x, wsc, idxsc).reshape(B, N_OUT)


def _combine_kernel(a_ref, b_ref, gamma_ref, beta_ref, co_ref, out_ref):
    h = jnp.maximum(a_ref[...] + b_ref[...], 0.0)
    mean = jnp.mean(h, axis=0, keepdims=True)
    d = h - mean
    var = jnp.mean(d * d, axis=0, keepdims=True)
    hn = d * jax.lax.rsqrt(var + 1e-5) * gamma_ref[...] + beta_ref[...]
    out_ref[...] = hn * jax.nn.sigmoid(co_ref[...])


def _combine(wx_tc, wx_sc, gamma2, beta2, co2):
    return pl.pallas_call(
        _combine_kernel,
        out_shape=jax.ShapeDtypeStruct((B, N_OUT), jnp.float32),
    )(wx_tc, wx_sc, gamma2, beta2, co2)


@jax.jit
def kernel(x, weight, gamma, beta, co_weight, idx):
    idx32 = idx.astype(jnp.int32)
    idx3 = idx32[:TC_COLS].reshape(NB, 1, KBLK)
    wsc = weight[0, TC_COLS:]
    idxsc = idx32[TC_COLS:]
    wx_tc = _tc_main(x, weight, idx3)
    wx_sc = _sc_partial(x, wsc, idxsc)
    return _combine(
        wx_tc,
        wx_sc,
        gamma.reshape(1, N_OUT),
        beta.reshape(1, N_OUT),
        co_weight.reshape(1, N_OUT),
    )


# speedup vs baseline: 4.9673x; 4.9673x over previous
import jax
import jax.numpy as jnp
from jax.experimental import pallas as pl
from jax.experimental.pallas import tpu as pltpu

N_IN = 16384
N_OUT = 128
B = 1024
KBLK = 2048
NB = N_IN // KBLK


def _spw_kernel(x_ref, w_ref, idx_ref, gamma_ref, beta_ref, co_ref, out_ref, acc_ref):
    k = pl.program_id(0)

    idxv = idx_ref[0, 0, :]
    onehot = jnp.where(
        idxv[:, None] == jax.lax.broadcasted_iota(jnp.int32, (KBLK, N_OUT), 1),
        w_ref[0, :][:, None],
        0.0,
    )
    contrib = jnp.dot(x_ref[...], onehot, preferred_element_type=jnp.float32)

    @pl.when(k == 0)
    def _init():
        acc_ref[...] = contrib

    @pl.when(k > 0)
    def _acc():
        acc_ref[...] += contrib

    @pl.when(k == NB - 1)
    def _finish():
        h = jnp.maximum(acc_ref[...], 0.0)
        mean = jnp.mean(h, axis=0, keepdims=True)
        d = h - mean
        var = jnp.mean(d * d, axis=0, keepdims=True)
        hn = d * jax.lax.rsqrt(var + 1e-5) * gamma_ref[...] + beta_ref[...]
        out_ref[...] = hn * jax.nn.sigmoid(co_ref[...])


@jax.jit
def kernel(x, weight, gamma, beta, co_weight, idx):
    idx3 = idx.astype(jnp.int32).reshape(NB, 1, KBLK)
    gamma2 = gamma.reshape(1, N_OUT)
    beta2 = beta.reshape(1, N_OUT)
    co2 = co_weight.reshape(1, N_OUT)
    return pl.pallas_call(
        _spw_kernel,
        grid=(NB,),
        in_specs=[
            pl.BlockSpec((B, KBLK), lambda k: (0, k)),
            pl.BlockSpec((1, KBLK), lambda k: (0, k)),
            pl.BlockSpec((1, 1, KBLK), lambda k: (k, 0, 0)),
            pl.BlockSpec((1, N_OUT), lambda k: (0, 0)),
            pl.BlockSpec((1, N_OUT), lambda k: (0, 0)),
            pl.BlockSpec((1, N_OUT), lambda k: (0, 0)),
        ],
        out_specs=pl.BlockSpec((B, N_OUT), lambda k: (0, 0)),
        out_shape=jax.ShapeDtypeStruct((B, N_OUT), jnp.float32),
        scratch_shapes=[pltpu.VMEM((B, N_OUT), jnp.float32)],
    )(x, weight, idx3, gamma2, beta2, co2)
